# baseline (speedup 1.0000x reference)
import jax
import jax.numpy as jnp
from jax.experimental import pallas as pl
from jax.experimental.pallas import tpu as pltpu

_ROWS_PER_BLOCK = 256
_BISECT_ITERS = 20


def _sparsemax_block(x_ref, o_ref):
    x = x_ref[...]
    d = x.shape[1]
    ones = jnp.ones((d, 1), dtype=x.dtype)
    m = jnp.max(x, axis=1, keepdims=True)
    lo = m - 1.0
    hi = m
    for _ in range(_BISECT_ITERS):
        mid = 0.5 * (lo + hi)
        r = jnp.maximum(x - mid, 0.0)
        s = jax.lax.dot(r, ones, preferred_element_type=jnp.float32, precision=jax.lax.Precision.HIGHEST)
        ge = s >= 1.0
        lo = jnp.where(ge, mid, lo)
        hi = jnp.where(ge, hi, mid)
    mask = (x > lo).astype(x.dtype)
    cnt = jax.lax.dot(mask, ones, preferred_element_type=jnp.float32, precision=jax.lax.Precision.HIGHEST)
    ssum = jax.lax.dot(mask * x, ones, preferred_element_type=jnp.float32, precision=jax.lax.Precision.HIGHEST)
    tau = (ssum - 1.0) / jnp.maximum(cnt, 1.0)
    o_ref[...] = jnp.maximum(x - tau, 0.0)


def kernel(input):
    b, s, d = input.shape
    n = b * s
    x2 = input.reshape(n, d)
    out = pl.pallas_call(
        _sparsemax_block,
        grid=(n // _ROWS_PER_BLOCK,),
        in_specs=[pl.BlockSpec((_ROWS_PER_BLOCK, d), lambda i: (i, 0))],
        out_specs=pl.BlockSpec((_ROWS_PER_BLOCK, d), lambda i: (i, 0)),
        out_shape=jax.ShapeDtypeStruct((n, d), input.dtype),
        compiler_params=pltpu.CompilerParams(
            dimension_semantics=("arbitrary",),
        ),
    )(x2)
    return out.reshape(b, s, d)


# speedup vs baseline: 19.2505x; 19.2505x over previous
import jax
import jax.numpy as jnp
from jax.experimental import pallas as pl
from jax.experimental.pallas import tpu as pltpu

_ROWS_PER_BLOCK = 256
_BISECT_ITERS = 16


def _sparsemax_block(x_ref, o_ref):
    x = x_ref[...]
    m = jnp.max(x, axis=1, keepdims=True)
    lo = m - 1.0
    hi = m
    for _ in range(_BISECT_ITERS):
        mid = 0.5 * (lo + hi)
        s = jnp.sum(jnp.maximum(x - mid, 0.0), axis=1, keepdims=True)
        ge = s >= 1.0
        lo = jnp.where(ge, mid, lo)
        hi = jnp.where(ge, hi, mid)
    r = jnp.maximum(x - lo, 0.0)
    cnt = jnp.sum((r > 0.0).astype(x.dtype), axis=1, keepdims=True)
    ssum = jnp.sum(r, axis=1, keepdims=True)
    delta = (ssum - 1.0) / jnp.maximum(cnt, 1.0)
    o_ref[...] = jnp.maximum(r - delta, 0.0)


def kernel(input):
    b, s, d = input.shape
    n = b * s
    x2 = input.reshape(n, d)
    out = pl.pallas_call(
        _sparsemax_block,
        grid=(n // _ROWS_PER_BLOCK,),
        in_specs=[pl.BlockSpec((_ROWS_PER_BLOCK, d), lambda i: (i, 0))],
        out_specs=pl.BlockSpec((_ROWS_PER_BLOCK, d), lambda i: (i, 0)),
        out_shape=jax.ShapeDtypeStruct((n, d), input.dtype),
        compiler_params=pltpu.CompilerParams(
            dimension_semantics=("arbitrary",),
        ),
    )(x2)
    return out.reshape(b, s, d)


# speedup vs baseline: 24.3006x; 1.2623x over previous
import jax
import jax.numpy as jnp
from jax.experimental import pallas as pl
from jax.experimental.pallas import tpu as pltpu

_ROWS_PER_BLOCK = 256
_BISECT_ITERS = 12


def _sparsemax_block(x_ref, o_ref):
    x = x_ref[...]
    m = jnp.max(x, axis=1, keepdims=True)
    lo = m - 1.0
    hi = m
    for _ in range(_BISECT_ITERS):
        mid = 0.5 * (lo + hi)
        s = jnp.sum(jnp.maximum(x - mid, 0.0), axis=1, keepdims=True)
        ge = s >= 1.0
        lo = jnp.where(ge, mid, lo)
        hi = jnp.where(ge, hi, mid)
    r = jnp.maximum(x - lo, 0.0)
    cnt = jnp.sum((r > 0.0).astype(x.dtype), axis=1, keepdims=True)
    ssum = jnp.sum(r, axis=1, keepdims=True)
    delta = (ssum - 1.0) / jnp.maximum(cnt, 1.0)
    o_ref[...] = jnp.maximum(r - delta, 0.0)


def kernel(input):
    b, s, d = input.shape
    n = b * s
    x2 = input.reshape(n, d)
    out = pl.pallas_call(
        _sparsemax_block,
        grid=(n // _ROWS_PER_BLOCK,),
        in_specs=[pl.BlockSpec((_ROWS_PER_BLOCK, d), lambda i: (i, 0))],
        out_specs=pl.BlockSpec((_ROWS_PER_BLOCK, d), lambda i: (i, 0)),
        out_shape=jax.ShapeDtypeStruct((n, d), input.dtype),
        compiler_params=pltpu.CompilerParams(
            dimension_semantics=("arbitrary",),
        ),
    )(x2)
    return out.reshape(b, s, d)


# speedup vs baseline: 32.9406x; 1.3555x over previous
import jax
import jax.numpy as jnp
from jax.experimental import pallas as pl
from jax.experimental.pallas import tpu as pltpu

_ROWS_PER_BLOCK = 256
_BISECT_ITERS = 4
_MICHELOT_ITERS = 3


def _sparsemax_block(x_ref, o_ref):
    x = x_ref[...]
    m = jnp.max(x, axis=1, keepdims=True)
    lo = m - 1.0
    hi = m
    for _ in range(_BISECT_ITERS):
        mid = 0.5 * (lo + hi)
        s = jnp.sum(jnp.maximum(x - mid, 0.0), axis=1, keepdims=True)
        ge = s >= 1.0
        lo = jnp.where(ge, mid, lo)
        hi = jnp.where(ge, hi, mid)
    t = lo
    for _ in range(_MICHELOT_ITERS - 1):
        r = jnp.maximum(x - t, 0.0)
        cnt = jnp.sum((r > 0.0).astype(x.dtype), axis=1, keepdims=True)
        ssum = jnp.sum(r, axis=1, keepdims=True)
        t = t + (ssum - 1.0) / jnp.maximum(cnt, 1.0)
    r = jnp.maximum(x - t, 0.0)
    cnt = jnp.sum((r > 0.0).astype(x.dtype), axis=1, keepdims=True)
    ssum = jnp.sum(r, axis=1, keepdims=True)
    delta = (ssum - 1.0) / jnp.maximum(cnt, 1.0)
    o_ref[...] = jnp.maximum(r - delta, 0.0)


def kernel(input):
    b, s, d = input.shape
    n = b * s
    x2 = input.reshape(n, d)
    out = pl.pallas_call(
        _sparsemax_block,
        grid=(n // _ROWS_PER_BLOCK,),
        in_specs=[pl.BlockSpec((_ROWS_PER_BLOCK, d), lambda i: (i, 0))],
        out_specs=pl.BlockSpec((_ROWS_PER_BLOCK, d), lambda i: (i, 0)),
        out_shape=jax.ShapeDtypeStruct((n, d), input.dtype),
        compiler_params=pltpu.CompilerParams(
            dimension_semantics=("arbitrary",),
        ),
    )(x2)
    return out.reshape(b, s, d)
